# initial kernel scaffold (unmeasured)
import jax
import jax.numpy as jnp
from jax import lax
from jax.experimental import pallas as pl
from jax.experimental.pallas import tpu as pltpu


def kernel(x, dy):
    k, d = x.shape
    _, f = dy.shape
    half = d // 2

    def body(x_ref, dy_ref, out_ref, send_ref, recv_ref, send_sem, recv_sem):
        my_x = lax.axis_index("x")
        my_y = lax.axis_index("y")
        my_z = lax.axis_index("z")
        other = 1 - my_y
        nbr = (my_x, other, my_z)

        barrier_sem = pltpu.get_barrier_semaphore()
        pl.semaphore_signal(
            barrier_sem, inc=1, device_id=nbr,
            device_id_type=pl.DeviceIdType.MESH,
        )
        pl.semaphore_wait(barrier_sem, 1)

        xb = x_ref[:, :].astype(jnp.bfloat16)
        dyb = dy_ref[:, :].astype(jnp.bfloat16)

        x_send = lax.dynamic_slice(xb, (0, other * half), (k, half))
        part_send = lax.dot_general(
            x_send, dyb, (((0,), (0,)), ((), ())),
            preferred_element_type=jnp.float32,
        )
        send_ref[:, :] = part_send.astype(jnp.bfloat16)

        rdma = pltpu.make_async_remote_copy(
            src_ref=send_ref,
            dst_ref=recv_ref,
            send_sem=send_sem,
            recv_sem=recv_sem,
            device_id=nbr,
            device_id_type=pl.DeviceIdType.MESH,
        )
        rdma.start()

        x_keep = lax.dynamic_slice(xb, (0, my_y * half), (k, half))
        part_keep = lax.dot_general(
            x_keep, dyb, (((0,), (0,)), ((), ())),
            preferred_element_type=jnp.float32,
        )

        rdma.wait()
        out_ref[:, :] = part_keep + recv_ref[:, :].astype(jnp.float32)

    return pl.pallas_call(
        body,
        out_shape=jax.ShapeDtypeStruct((half, f), jnp.float32),
        in_specs=[
            pl.BlockSpec(memory_space=pltpu.VMEM),
            pl.BlockSpec(memory_space=pltpu.VMEM),
        ],
        out_specs=pl.BlockSpec(memory_space=pltpu.VMEM),
        scratch_shapes=[
            pltpu.VMEM((half, f), jnp.bfloat16),
            pltpu.VMEM((half, f), jnp.bfloat16),
            pltpu.SemaphoreType.DMA,
            pltpu.SemaphoreType.DMA,
        ],
        compiler_params=pltpu.CompilerParams(collective_id=0),
    )(x, dy)


# baseline (device time: 21157 ns/iter reference)
import jax
import jax.numpy as jnp
from jax import lax
from jax.experimental import pallas as pl
from jax.experimental.pallas import tpu as pltpu


def kernel(x, dy):
    k, d = x.shape
    _, f = dy.shape
    half = d // 2

    def body(x_ref, dy_ref, out_ref, send_ref, recv_ref, send_sem, recv_sem):
        my_x = lax.axis_index("x")
        my_y = lax.axis_index("y")
        my_z = lax.axis_index("z")
        other = 1 - my_y
        nbr = (my_x, other, my_z)

        barrier_sem = pltpu.get_barrier_semaphore()
        pl.semaphore_signal(
            barrier_sem, inc=1, device_id=nbr,
            device_id_type=pl.DeviceIdType.MESH,
        )
        pl.semaphore_wait(barrier_sem, 1)

        dyb = dy_ref[:, :].astype(jnp.bfloat16)

        x_send = x_ref[:, pl.ds(other * half, half)].astype(jnp.bfloat16)
        part_send = lax.dot_general(
            x_send, dyb, (((0,), (0,)), ((), ())),
            preferred_element_type=jnp.float32,
        )
        send_ref[:, :] = part_send.astype(jnp.bfloat16)

        rdma = pltpu.make_async_remote_copy(
            src_ref=send_ref,
            dst_ref=recv_ref,
            send_sem=send_sem,
            recv_sem=recv_sem,
            device_id=nbr,
            device_id_type=pl.DeviceIdType.MESH,
        )
        rdma.start()

        x_keep = x_ref[:, pl.ds(my_y * half, half)].astype(jnp.bfloat16)
        part_keep = lax.dot_general(
            x_keep, dyb, (((0,), (0,)), ((), ())),
            preferred_element_type=jnp.float32,
        )

        rdma.wait()
        out_ref[:, :] = part_keep + recv_ref[:, :].astype(jnp.float32)

    return pl.pallas_call(
        body,
        out_shape=jax.ShapeDtypeStruct((half, f), jnp.float32),
        in_specs=[
            pl.BlockSpec(memory_space=pltpu.VMEM),
            pl.BlockSpec(memory_space=pltpu.VMEM),
        ],
        out_specs=pl.BlockSpec(memory_space=pltpu.VMEM),
        scratch_shapes=[
            pltpu.VMEM((half, f), jnp.bfloat16),
            pltpu.VMEM((half, f), jnp.bfloat16),
            pltpu.SemaphoreType.DMA,
            pltpu.SemaphoreType.DMA,
        ],
        compiler_params=pltpu.CompilerParams(collective_id=0),
    )(x, dy)
